# baseline (speedup 1.0000x reference)
import jax
import jax.numpy as jnp
from jax.experimental import pallas as pl
from jax.experimental.pallas import tpu as pltpu

_B, _S, _D, _L = 1, 512, 768, 16
_EPB = 2
_NSTEP = _L // _EPB


def _hyper_lattice_kernel(x_ref, gate_w_ref, w_ref, out_w_ref, out_b_ref,
                          ln_g_ref, ln_b_ref, out_ref, acc_ref, idx_ref,
                          xcat_ref):
    s = pl.program_id(0)
    x = x_ref[...]

    @pl.when(s == 0)
    def _route():
        logits = jnp.dot(x, gate_w_ref[...].T,
                         preferred_element_type=jnp.float32)
        idx_ref[...] = jnp.argmax(logits, axis=-1, keepdims=True).astype(
            jnp.int32)

    for j in range(_EPB):
        e = s * _EPB + j
        xcat_ref[:, j * _D:(j + 1) * _D] = jnp.where(
            idx_ref[...] == e, x, 0.0).astype(jnp.bfloat16)

    wcat = w_ref[...].reshape(_EPB * _D, _D).astype(jnp.bfloat16)
    c = jnp.dot(xcat_ref[...], wcat, preferred_element_type=jnp.float32)
    proj = jnp.dot(c.astype(jnp.bfloat16),
                   out_w_ref[...].T.astype(jnp.bfloat16),
                   preferred_element_type=jnp.float32)

    @pl.when(s == 0)
    def _first():
        acc_ref[...] = proj

    @pl.when(s > 0)
    def _accum():
        acc_ref[...] += proj

    @pl.when(s == _NSTEP - 1)
    def _epilogue():
        h = x + acc_ref[...] + out_b_ref[...]
        mu = jnp.mean(h, axis=-1, keepdims=True)
        var = jnp.mean((h - mu) ** 2, axis=-1, keepdims=True)
        out_ref[...] = ((h - mu) * jax.lax.rsqrt(var + 1e-5)
                        * ln_g_ref[...] + ln_b_ref[...])


def kernel(x, gate_w, lattice_weights, out_w, out_b, ln_g, ln_b):
    x2 = x.reshape(_S, _D)
    out = pl.pallas_call(
        _hyper_lattice_kernel,
        grid=(_NSTEP,),
        in_specs=[
            pl.BlockSpec((_S, _D), lambda s: (0, 0)),
            pl.BlockSpec((_L, _D), lambda s: (0, 0)),
            pl.BlockSpec((_EPB, _D, _D), lambda s: (s, 0, 0)),
            pl.BlockSpec((_D, _D), lambda s: (0, 0)),
            pl.BlockSpec((1, _D), lambda s: (0, 0)),
            pl.BlockSpec((1, _D), lambda s: (0, 0)),
            pl.BlockSpec((1, _D), lambda s: (0, 0)),
        ],
        out_specs=pl.BlockSpec((_S, _D), lambda s: (0, 0)),
        out_shape=jax.ShapeDtypeStruct((_S, _D), jnp.float32),
        scratch_shapes=[
            pltpu.VMEM((_S, _D), jnp.float32),
            pltpu.VMEM((_S, 1), jnp.int32),
            pltpu.VMEM((_S, _EPB * _D), jnp.bfloat16),
        ],
    )(x2, gate_w, lattice_weights, out_w,
      out_b.reshape(1, _D), ln_g.reshape(1, _D), ln_b.reshape(1, _D))
    return out.reshape(_B, _S, _D)


# speedup vs baseline: 1.0069x; 1.0069x over previous
import jax
import jax.numpy as jnp
from jax.experimental import pallas as pl
from jax.experimental.pallas import tpu as pltpu

_B, _S, _D, _L = 1, 512, 768, 16
_EPB = 4
_NSTEP = _L // _EPB


_H = _D // 2


def _hyper_lattice_kernel(x_ref, gate_w_ref, w_ref, out_w_ref, out_b_ref,
                          ln_g_ref, ln_b_ref, out_ref, acc_ref, idx_ref,
                          xcat_ref):
    s = pl.program_id(0)
    n = pl.program_id(1)
    x = x_ref[...]

    @pl.when((s == 0) & (n == 0))
    def _route():
        logits = jnp.dot(x, gate_w_ref[...].T,
                         preferred_element_type=jnp.float32)
        idx_ref[...] = jnp.argmax(logits, axis=-1, keepdims=True).astype(
            jnp.int32)

    @pl.when(n == 0)
    def _build_xcat():
        for j in range(_EPB):
            e = s * _EPB + j
            xcat_ref[:, j * _D:(j + 1) * _D] = jnp.where(
                idx_ref[...] == e, x, 0.0).astype(jnp.bfloat16)

    wcat = w_ref[...].reshape(_EPB * _D, _H).astype(jnp.bfloat16)
    c = jnp.dot(xcat_ref[...], wcat, preferred_element_type=jnp.float32)

    @pl.when(s == 0)
    def _first():
        acc_ref[:, pl.ds(n * _H, _H)] = c

    @pl.when(s > 0)
    def _accum():
        acc_ref[:, pl.ds(n * _H, _H)] += c

    @pl.when((s == _NSTEP - 1) & (n == 1))
    def _epilogue():
        out2 = jnp.dot(acc_ref[...].astype(jnp.bfloat16),
                       out_w_ref[...].T.astype(jnp.bfloat16),
                       preferred_element_type=jnp.float32) + out_b_ref[...]
        h = x + out2
        mu = jnp.mean(h, axis=-1, keepdims=True)
        var = jnp.mean((h - mu) ** 2, axis=-1, keepdims=True)
        out_ref[...] = ((h - mu) * jax.lax.rsqrt(var + 1e-5)
                        * ln_g_ref[...] + ln_b_ref[...])


def kernel(x, gate_w, lattice_weights, out_w, out_b, ln_g, ln_b):
    x2 = x.reshape(_S, _D)
    out = pl.pallas_call(
        _hyper_lattice_kernel,
        grid=(_NSTEP, 2),
        in_specs=[
            pl.BlockSpec((_S, _D), lambda s, n: (0, 0)),
            pl.BlockSpec((_L, _D), lambda s, n: (0, 0)),
            pl.BlockSpec((_EPB, _D, _H), lambda s, n: (s, 0, n)),
            pl.BlockSpec((_D, _D), lambda s, n: (0, 0)),
            pl.BlockSpec((1, _D), lambda s, n: (0, 0)),
            pl.BlockSpec((1, _D), lambda s, n: (0, 0)),
            pl.BlockSpec((1, _D), lambda s, n: (0, 0)),
        ],
        out_specs=pl.BlockSpec((_S, _D), lambda s, n: (0, 0)),
        out_shape=jax.ShapeDtypeStruct((_S, _D), jnp.float32),
        scratch_shapes=[
            pltpu.VMEM((_S, _D), jnp.float32),
            pltpu.VMEM((_S, 1), jnp.int32),
            pltpu.VMEM((_S, _EPB * _D), jnp.bfloat16),
        ],
    )(x2, gate_w, lattice_weights, out_w,
      out_b.reshape(1, _D), ln_g.reshape(1, _D), ln_b.reshape(1, _D))
    return out.reshape(_B, _S, _D)


# speedup vs baseline: 1.1625x; 1.1545x over previous
import jax
import jax.numpy as jnp
from jax.experimental import pallas as pl
from jax.experimental.pallas import tpu as pltpu

_B, _S, _D, _L = 1, 512, 768, 16
_EPB = 4
_NSTEP = _L // _EPB


def _hyper_lattice_kernel(x_ref, gate_w_ref, w_ref, out_w_ref, out_b_ref,
                          ln_g_ref, ln_b_ref, out_ref, acc_ref, xcat_ref):
    s = pl.program_id(0)
    x = x_ref[...]

    @pl.when(s == 0)
    def _route_and_mask():
        logits = jnp.dot(x, gate_w_ref[...].T,
                         preferred_element_type=jnp.float32)
        idx = jnp.argmax(logits, axis=-1, keepdims=True).astype(jnp.int32)
        for e in range(_L):
            xcat_ref[:, e * _D:(e + 1) * _D] = jnp.where(
                idx == e, x, 0.0).astype(jnp.bfloat16)

    wcat = w_ref[...].reshape(_EPB * _D, _D).astype(jnp.bfloat16)
    c = jnp.dot(xcat_ref[:, pl.ds(s * _EPB * _D, _EPB * _D)], wcat,
                preferred_element_type=jnp.float32)

    @pl.when(s == 0)
    def _first():
        acc_ref[...] = c

    @pl.when(s > 0)
    def _accum():
        acc_ref[...] += c

    @pl.when(s == _NSTEP - 1)
    def _epilogue():
        out2 = jnp.dot(acc_ref[...].astype(jnp.bfloat16),
                       out_w_ref[...].T.astype(jnp.bfloat16),
                       preferred_element_type=jnp.float32) + out_b_ref[...]
        h = x + out2
        mu = jnp.mean(h, axis=-1, keepdims=True)
        var = jnp.mean((h - mu) ** 2, axis=-1, keepdims=True)
        out_ref[...] = ((h - mu) * jax.lax.rsqrt(var + 1e-5)
                        * ln_g_ref[...] + ln_b_ref[...])


def kernel(x, gate_w, lattice_weights, out_w, out_b, ln_g, ln_b):
    x2 = x.reshape(_S, _D)
    out = pl.pallas_call(
        _hyper_lattice_kernel,
        grid=(_NSTEP,),
        in_specs=[
            pl.BlockSpec((_S, _D), lambda s: (0, 0)),
            pl.BlockSpec((_L, _D), lambda s: (0, 0)),
            pl.BlockSpec((_EPB, _D, _D), lambda s: (s, 0, 0)),
            pl.BlockSpec((_D, _D), lambda s: (0, 0)),
            pl.BlockSpec((1, _D), lambda s: (0, 0)),
            pl.BlockSpec((1, _D), lambda s: (0, 0)),
            pl.BlockSpec((1, _D), lambda s: (0, 0)),
        ],
        out_specs=pl.BlockSpec((_S, _D), lambda s: (0, 0)),
        out_shape=jax.ShapeDtypeStruct((_S, _D), jnp.float32),
        scratch_shapes=[
            pltpu.VMEM((_S, _D), jnp.float32),
            pltpu.VMEM((_S, _L * _D), jnp.bfloat16),
        ],
    )(x2, gate_w, lattice_weights, out_w,
      out_b.reshape(1, _D), ln_g.reshape(1, _D), ln_b.reshape(1, _D))
    return out.reshape(_B, _S, _D)


# speedup vs baseline: 1.2199x; 1.0494x over previous
import jax
import jax.numpy as jnp
from jax.experimental import pallas as pl
from jax.experimental.pallas import tpu as pltpu

_B, _S, _D, _L = 1, 512, 768, 16
_EPB = 4
_NSTEP = _L // _EPB


def _hyper_lattice_kernel(x_ref, gate_w_ref, w_ref, out_w_ref, out_b_ref,
                          ln_g_ref, ln_b_ref, out_ref, acc_ref, idx_ref,
                          xcat_ref):
    s = pl.program_id(0)
    x = x_ref[...]

    @pl.when(s == 0)
    def _route():
        logits = jnp.dot(x, gate_w_ref[...].T,
                         preferred_element_type=jnp.float32)
        idx_ref[...] = jnp.argmax(logits, axis=-1, keepdims=True).astype(
            jnp.int32)

    for j in range(_EPB):
        e = s * _EPB + j
        xcat_ref[:, j * _D:(j + 1) * _D] = jnp.where(
            idx_ref[...] == e, x, 0.0).astype(jnp.bfloat16)

    wcat = w_ref[...].reshape(_EPB * _D, _D).astype(jnp.bfloat16)
    c = jnp.dot(xcat_ref[...], wcat, preferred_element_type=jnp.float32)

    @pl.when(s == 0)
    def _first():
        acc_ref[...] = c

    @pl.when(s > 0)
    def _accum():
        acc_ref[...] += c

    @pl.when(s == _NSTEP - 1)
    def _epilogue():
        out2 = jnp.dot(acc_ref[...].astype(jnp.bfloat16),
                       out_w_ref[...].T.astype(jnp.bfloat16),
                       preferred_element_type=jnp.float32) + out_b_ref[...]
        h = x + out2
        mu = jnp.mean(h, axis=-1, keepdims=True)
        var = jnp.mean((h - mu) ** 2, axis=-1, keepdims=True)
        out_ref[...] = ((h - mu) * jax.lax.rsqrt(var + 1e-5)
                        * ln_g_ref[...] + ln_b_ref[...])


def kernel(x, gate_w, lattice_weights, out_w, out_b, ln_g, ln_b):
    x2 = x.reshape(_S, _D)
    out = pl.pallas_call(
        _hyper_lattice_kernel,
        grid=(_NSTEP,),
        in_specs=[
            pl.BlockSpec((_S, _D), lambda s: (0, 0)),
            pl.BlockSpec((_L, _D), lambda s: (0, 0)),
            pl.BlockSpec((_EPB, _D, _D), lambda s: (s, 0, 0)),
            pl.BlockSpec((_D, _D), lambda s: (0, 0)),
            pl.BlockSpec((1, _D), lambda s: (0, 0)),
            pl.BlockSpec((1, _D), lambda s: (0, 0)),
            pl.BlockSpec((1, _D), lambda s: (0, 0)),
        ],
        out_specs=pl.BlockSpec((_S, _D), lambda s: (0, 0)),
        out_shape=jax.ShapeDtypeStruct((_S, _D), jnp.float32),
        scratch_shapes=[
            pltpu.VMEM((_S, _D), jnp.float32),
            pltpu.VMEM((_S, 1), jnp.int32),
            pltpu.VMEM((_S, _EPB * _D), jnp.bfloat16),
        ],
    )(x2, gate_w, lattice_weights, out_w,
      out_b.reshape(1, _D), ln_g.reshape(1, _D), ln_b.reshape(1, _D))
    return out.reshape(_B, _S, _D)
